# baseline (speedup 1.0000x reference)
import functools

import jax
import jax.numpy as jnp
from jax import lax
from jax.experimental import pallas as pl
from jax.experimental.pallas import tpu as pltpu
from jax.experimental.pallas import tpu_sc as plsc

H = 1024
E = 8
TOPK = 2
EPS = 1e-06
N = 2048
A = N * TOPK
TM = 128
T = 40
P = T * TM
NW = 32
GPW = P // NW
GC = GPW // 4
CPW = A // NW
CC = CPW // 4


def _router_body(x_ref, wg_ref, bg_ref, i12_ref, w12_ref, aux_ref):
    n = x_ref.shape[0]
    logits = lax.dot_general(
        x_ref[...], wg_ref[...], (((1,), (1,)), ((), ())),
        precision=lax.Precision.DEFAULT,
        preferred_element_type=jnp.float32) + bg_ref[...][None, :]
    m = jnp.max(logits, axis=1, keepdims=True)
    ex = jnp.exp(logits - m)
    probs = ex / jnp.sum(ex, axis=1, keepdims=True)
    iota = lax.broadcasted_iota(jnp.int32, (n, E), 1)
    p1 = jnp.max(probs, axis=1, keepdims=True)
    i1 = jnp.min(jnp.where(probs == p1, iota, E), axis=1, keepdims=True)
    masked = jnp.where(iota == i1, -jnp.inf, probs)
    p2 = jnp.max(masked, axis=1, keepdims=True)
    i2 = jnp.min(jnp.where(masked == p2, iota, E), axis=1, keepdims=True)
    denom = p1 + p2 + EPS
    i12_ref[...] = jnp.concatenate([i1, i2], axis=1)
    w12_ref[...] = jnp.concatenate([p1 / denom, p2 / denom], axis=1)
    mask = ((iota == i1) | (iota == i2)).astype(jnp.float32)
    usage = jnp.mean(mask, axis=0)
    gates = jnp.mean(probs, axis=0)
    aux_ref[0, 0] = jnp.sum(usage * gates) * E


def _mm_body(eot_ref, nact_ref, xg_ref, we_ref, be_ref, pwt_ref, buf_ref):
    t = pl.program_id(0)

    @pl.when(t < nact_ref[0])
    def _():
        y = lax.dot_general(
            xg_ref[...].astype(jnp.bfloat16), we_ref[0].astype(jnp.bfloat16),
            (((1,), (1,)), ((), ())),
            preferred_element_type=jnp.float32) + be_ref[0]
        buf_ref[...] = pwt_ref[...] * y


def _sc_gather_body(x_hbm, ptok_hbm, out_hbm, idx_v, b0_v, b1_v,
                    sg0, sg1, ss0, ss1):
    wid = lax.axis_index("s") * 2 + lax.axis_index("c")
    base = wid * GPW
    pltpu.sync_copy(ptok_hbm.at[pl.ds(base, GPW)], idx_v)
    g0 = pltpu.async_copy(x_hbm.at[idx_v.at[pl.ds(0, GC)]], b0_v, sg0)
    g1 = pltpu.async_copy(x_hbm.at[idx_v.at[pl.ds(GC, GC)]], b1_v, sg1)
    g0.wait()
    s0 = pltpu.async_copy(b0_v, out_hbm.at[pl.ds(base, GC)], ss0)
    g1.wait()
    s1 = pltpu.async_copy(b1_v, out_hbm.at[pl.ds(base + GC, GC)], ss1)
    s0.wait()
    g2 = pltpu.async_copy(x_hbm.at[idx_v.at[pl.ds(2 * GC, GC)]], b0_v, sg0)
    s1.wait()
    g3 = pltpu.async_copy(x_hbm.at[idx_v.at[pl.ds(3 * GC, GC)]], b1_v, sg1)
    g2.wait()
    s2 = pltpu.async_copy(b0_v, out_hbm.at[pl.ds(base + 2 * GC, GC)], ss0)
    g3.wait()
    s3 = pltpu.async_copy(b1_v, out_hbm.at[pl.ds(base + 3 * GC, GC)], ss1)
    s2.wait()
    s3.wait()


def _sc_combine_body(buf_hbm, pos_hbm, out_hbm, idx_v, b0_v, b1_v,
                     sg0, sg1, ss0, ss1):
    wid = lax.axis_index("s") * 2 + lax.axis_index("c")
    base = wid * CPW
    pltpu.sync_copy(pos_hbm.at[pl.ds(base, CPW)], idx_v)
    g0 = pltpu.async_copy(buf_hbm.at[idx_v.at[pl.ds(0, CC)]], b0_v, sg0)
    g1 = pltpu.async_copy(buf_hbm.at[idx_v.at[pl.ds(CC, CC)]], b1_v, sg1)
    g0.wait()
    s0 = pltpu.async_copy(b0_v, out_hbm.at[pl.ds(base, CC)], ss0)
    g1.wait()
    s1 = pltpu.async_copy(b1_v, out_hbm.at[pl.ds(base + CC, CC)], ss1)
    s0.wait()
    g2 = pltpu.async_copy(buf_hbm.at[idx_v.at[pl.ds(2 * CC, CC)]], b0_v, sg0)
    s1.wait()
    g3 = pltpu.async_copy(buf_hbm.at[idx_v.at[pl.ds(3 * CC, CC)]], b1_v, sg1)
    g2.wait()
    s2 = pltpu.async_copy(b0_v, out_hbm.at[pl.ds(base + 2 * CC, CC)], ss0)
    g3.wait()
    s3 = pltpu.async_copy(b1_v, out_hbm.at[pl.ds(base + 3 * CC, CC)], ss1)
    s2.wait()
    s3.wait()


def _add_body(g_ref, out_ref):
    out_ref[...] = g_ref[:, 0] + g_ref[:, 1]


@jax.jit
def kernel(x, Wg, bg, We, be):
    b, s, h = x.shape
    x_flat = x.reshape(-1, h)

    i12, w12, aux = pl.pallas_call(
        _router_body,
        in_specs=[
            pl.BlockSpec((N, h), lambda: (0, 0)),
            pl.BlockSpec((E, h), lambda: (0, 0)),
            pl.BlockSpec((E,), lambda: (0,)),
        ],
        out_specs=[
            pl.BlockSpec((N, TOPK), lambda: (0, 0)),
            pl.BlockSpec((N, TOPK), lambda: (0, 0)),
            pl.BlockSpec(memory_space=pltpu.SMEM),
        ],
        out_shape=[
            jax.ShapeDtypeStruct((N, TOPK), jnp.int32),
            jax.ShapeDtypeStruct((N, TOPK), jnp.float32),
            jax.ShapeDtypeStruct((1, 1), jnp.float32),
        ],
    )(x_flat, Wg, bg)

    flat_e = i12.reshape(A)
    oh = (flat_e[:, None] == jnp.arange(E)[None, :]).astype(jnp.int32)
    cum = jnp.cumsum(oh, axis=0)
    rank = jnp.take_along_axis(cum, flat_e[:, None], 1)[:, 0] - 1
    counts = cum[-1]
    tiles_pe = (counts + TM - 1) // TM
    cum_tiles = jnp.cumsum(tiles_pe)
    tile_off = cum_tiles - tiles_pe
    pos = tile_off[flat_e] * TM + rank
    ptok = jnp.zeros((P,), jnp.int32).at[pos].set(jnp.arange(A) // TOPK)
    pwt = jnp.zeros((P, 1), jnp.float32).at[pos, 0].set(w12.reshape(A))
    eot = jnp.minimum(
        jnp.sum(jnp.arange(T)[:, None] >= cum_tiles[None, :], axis=1),
        E - 1).astype(jnp.int32)
    nact = cum_tiles[-1:].astype(jnp.int32)

    mesh = plsc.VectorSubcoreMesh(core_axis_name="c", subcore_axis_name="s")
    xg = pl.kernel(
        _sc_gather_body,
        out_type=jax.ShapeDtypeStruct((P, h), jnp.float32),
        mesh=mesh,
        scratch_types=[
            pltpu.VMEM((GPW,), jnp.int32),
            pltpu.VMEM((GC, h), jnp.float32),
            pltpu.VMEM((GC, h), jnp.float32),
            pltpu.SemaphoreType.DMA,
            pltpu.SemaphoreType.DMA,
            pltpu.SemaphoreType.DMA,
            pltpu.SemaphoreType.DMA,
        ],
    )(x_flat, ptok)

    buf = pl.pallas_call(
        _mm_body,
        grid_spec=pltpu.PrefetchScalarGridSpec(
            num_scalar_prefetch=2,
            grid=(T,),
            in_specs=[
                pl.BlockSpec((TM, h), lambda t, eot, nact: (t, 0)),
                pl.BlockSpec((1, h, h), lambda t, eot, nact: (eot[t], 0, 0)),
                pl.BlockSpec((1, 1, h), lambda t, eot, nact: (eot[t], 0, 0)),
                pl.BlockSpec((TM, 1), lambda t, eot, nact: (t, 0)),
            ],
            out_specs=pl.BlockSpec((TM, h), lambda t, eot, nact: (t, 0)),
        ),
        out_shape=jax.ShapeDtypeStruct((P, h), jnp.float32),
    )(eot, nact, xg, We, be.reshape(E, 1, h), pwt)

    garr = pl.kernel(
        _sc_combine_body,
        out_type=jax.ShapeDtypeStruct((A, h), jnp.float32),
        mesh=mesh,
        scratch_types=[
            pltpu.VMEM((CPW,), jnp.int32),
            pltpu.VMEM((CC, h), jnp.float32),
            pltpu.VMEM((CC, h), jnp.float32),
            pltpu.SemaphoreType.DMA,
            pltpu.SemaphoreType.DMA,
            pltpu.SemaphoreType.DMA,
            pltpu.SemaphoreType.DMA,
        ],
    )(buf, pos)

    out = pl.pallas_call(
        _add_body,
        grid=(8,),
        in_specs=[
            pl.BlockSpec((N // 8, TOPK, h), lambda i: (i, 0, 0)),
        ],
        out_specs=pl.BlockSpec((N // 8, h), lambda i: (i, 0)),
        out_shape=jax.ShapeDtypeStruct((N, h), jnp.float32),
    )(garr.reshape(N, TOPK, h))

    return out.reshape(b, s, h), aux[0, 0]


# speedup vs baseline: 4.0328x; 4.0328x over previous
import functools

import jax
import jax.numpy as jnp
from jax.experimental import pallas as pl
from jax.experimental.pallas import tpu as pltpu

H = 1024
E = 8
TOPK = 2
EPS = 1e-06


def _moe_body(x_ref, wg_ref, bg_ref, we_ref, be_ref,
              out_ref, aux_ref, w1_ref, w2_ref, i1_ref, i2_ref, xb_ref):
    e = pl.program_id(0)
    n = x_ref.shape[0]

    @pl.when(e == 0)
    def _router():
        xb_ref[...] = x_ref[...].astype(jnp.bfloat16)
        logits = jax.lax.dot_general(
            x_ref[...], wg_ref[...], (((1,), (1,)), ((), ())),
            precision=jax.lax.Precision.DEFAULT,
            preferred_element_type=jnp.float32) + bg_ref[...][None, :]
        m = jnp.max(logits, axis=1, keepdims=True)
        ex = jnp.exp(logits - m)
        probs = ex / jnp.sum(ex, axis=1, keepdims=True)
        iota = jax.lax.broadcasted_iota(jnp.int32, (n, E), 1)
        p1 = jnp.max(probs, axis=1, keepdims=True)
        i1 = jnp.min(jnp.where(probs == p1, iota, E), axis=1, keepdims=True)
        masked = jnp.where(iota == i1, -jnp.inf, probs)
        p2 = jnp.max(masked, axis=1, keepdims=True)
        i2 = jnp.min(jnp.where(masked == p2, iota, E), axis=1, keepdims=True)
        denom = p1 + p2 + EPS
        w1_ref[...] = p1 / denom
        w2_ref[...] = p2 / denom
        i1_ref[...] = i1
        i2_ref[...] = i2
        mask = ((iota == i1) | (iota == i2)).astype(jnp.float32)
        usage = jnp.mean(mask, axis=0)
        gates = jnp.mean(probs, axis=0)
        aux_ref[0, 0] = jnp.sum(usage * gates) * E

    w_col = (jnp.where(i1_ref[...] == e, w1_ref[...], 0.0)
             + jnp.where(i2_ref[...] == e, w2_ref[...], 0.0))

    web = we_ref[0].astype(jnp.bfloat16)
    y = jax.lax.dot_general(
        xb_ref[...], web, (((1,), (1,)), ((), ())),
        preferred_element_type=jnp.float32) + be_ref[0]
    contrib = w_col * y

    @pl.when(e == 0)
    def _init():
        out_ref[...] = contrib

    @pl.when(e > 0)
    def _acc():
        out_ref[...] += contrib


@jax.jit
def kernel(x, Wg, bg, We, be):
    b, s, h = x.shape
    x_flat = x.reshape(-1, h)
    n = x_flat.shape[0]

    out, aux = pl.pallas_call(
        _moe_body,
        grid=(E,),
        in_specs=[
            pl.BlockSpec((n, h), lambda e: (0, 0)),
            pl.BlockSpec((E, h), lambda e: (0, 0)),
            pl.BlockSpec((E,), lambda e: (0,)),
            pl.BlockSpec((1, h, h), lambda e: (e, 0, 0)),
            pl.BlockSpec((1, 1, h), lambda e: (e, 0, 0)),
        ],
        out_specs=[
            pl.BlockSpec((n, h), lambda e: (0, 0)),
            pl.BlockSpec(memory_space=pltpu.SMEM),
        ],
        out_shape=[
            jax.ShapeDtypeStruct((n, h), jnp.float32),
            jax.ShapeDtypeStruct((1, 1), jnp.float32),
        ],
        scratch_shapes=[
            pltpu.VMEM((n, 1), jnp.float32),
            pltpu.VMEM((n, 1), jnp.float32),
            pltpu.VMEM((n, 1), jnp.int32),
            pltpu.VMEM((n, 1), jnp.int32),
            pltpu.VMEM((n, h), jnp.bfloat16),
        ],
    )(x_flat, Wg, bg, We, be.reshape(E, 1, h))

    return out.reshape(b, s, h), aux[0, 0]
